# Initial kernel scaffold; baseline (speedup 1.0000x reference)
import jax
import jax.numpy as jnp
from jax.experimental import pallas as pl


def kernel(batch_factors, emb, W, b, gamma, beta):
    raise NotImplementedError("write your pallas kernel here")


# speedup vs baseline: 7.7374x; 7.7374x over previous
import functools
import math

import jax
import jax.numpy as jnp
from jax import lax
from jax.experimental import pallas as pl
from jax.experimental.pallas import tpu as pltpu
from jax.experimental.pallas import tpu_sc as plsc

F = 26
VOCAB = 100000
D = 32
D_OUT = 128
B = 16384

_ROWS = B * F
_NW = 32
_ROWS_PER_W = _ROWS // _NW
_GCHUNK = 128
_NG = _ROWS_PER_W // _GCHUNK


def _sc_gather_body(idx_hbm, table_hbm, out_hbm, idx_v, rows_v, gsem, osem):
    nc = 2
    wid = lax.axis_index("s") * nc + lax.axis_index("c")
    pltpu.sync_copy(idx_hbm.at[pl.ds(wid * _NG, _NG)], idx_v)

    nbuf = 4

    def chunk(g, _):
        for t in range(nbuf):
            j = g * nbuf + t
            pltpu.async_copy(table_hbm.at[idx_v.at[j]], rows_v.at[t], gsem).wait()
            row_base = wid * _ROWS_PER_W + j * _GCHUNK
            pltpu.async_copy(rows_v.at[t], out_hbm.at[pl.ds(row_base, _GCHUNK)],
                             osem).wait()

    lax.fori_loop(0, _NG // nbuf, chunk, None)


@functools.partial(
    pl.kernel,
    mesh=plsc.VectorSubcoreMesh(core_axis_name="c", subcore_axis_name="s"),
    out_type=jax.ShapeDtypeStruct((_ROWS, D), jnp.float32),
    scratch_types=[
        pltpu.VMEM((_NG, _GCHUNK), jnp.int32),
        pltpu.VMEM((4, _GCHUNK, D), jnp.float32),
        pltpu.SemaphoreType.DMA,
        pltpu.SemaphoreType.DMA,
    ],
    compiler_params=pltpu.CompilerParams(use_tc_tiling_on_sc=False),
)
def _sc_gather(idx_hbm, table_hbm, out_hbm, idx_v, rows_v, gsem, osem):
    _sc_gather_body(idx_hbm, table_hbm, out_hbm, idx_v, rows_v, gsem, osem)


_BM = 2048


def _tc_mlp_body(x_ref, w_ref, b_ref, g_ref, be_ref, o_ref):
    x = x_ref[...]
    y = jnp.dot(x, w_ref[...], preferred_element_type=jnp.float32) + b_ref[...]
    mu = jnp.mean(y, axis=-1, keepdims=True)
    var = jnp.mean((y - mu) * (y - mu), axis=-1, keepdims=True)
    y = (y - mu) * lax.rsqrt(var + 1e-5)
    y = y * g_ref[...] + be_ref[...]
    o_ref[...] = y * 0.5 * (1.0 + lax.erf(y * (1.0 / math.sqrt(2.0))))


def _tc_mlp(x, W, b, gamma, beta):
    grid = (B // _BM,)
    return pl.pallas_call(
        _tc_mlp_body,
        grid=grid,
        in_specs=[
            pl.BlockSpec((_BM, F * D), lambda i: (i, 0)),
            pl.BlockSpec((F * D, D_OUT), lambda i: (0, 0)),
            pl.BlockSpec((1, D_OUT), lambda i: (0, 0)),
            pl.BlockSpec((1, D_OUT), lambda i: (0, 0)),
            pl.BlockSpec((1, D_OUT), lambda i: (0, 0)),
        ],
        out_specs=pl.BlockSpec((_BM, D_OUT), lambda i: (i, 0)),
        out_shape=jax.ShapeDtypeStruct((B, D_OUT), jnp.float32),
    )(x, W, b, gamma, beta)


def kernel(batch_factors, emb, W, b, gamma, beta):
    offs = (jnp.arange(F, dtype=jnp.int32) * VOCAB)[None, :]
    flat_idx = (batch_factors + offs).reshape(_ROWS // _GCHUNK, _GCHUNK)
    table = emb.reshape(F * VOCAB, D)
    gathered = _sc_gather(flat_idx, table)
    x = gathered.reshape(B, F * D)
    out = _tc_mlp(x, W, b.reshape(1, D_OUT), gamma.reshape(1, D_OUT),
                  beta.reshape(1, D_OUT))
    return (out, jnp.ones((F,), dtype=jnp.float32))


# speedup vs baseline: 8.1659x; 1.0554x over previous
import functools
import math

import jax
import jax.numpy as jnp
from jax import lax
from jax.experimental import pallas as pl
from jax.experimental.pallas import tpu as pltpu
from jax.experimental.pallas import tpu_sc as plsc

F = 26
VOCAB = 100000
D = 32
D_OUT = 128
B = 16384

_ROWS = B * F
_NW = 32
_ROWS_PER_W = _ROWS // _NW
_GCHUNK = 128
_NG = _ROWS_PER_W // _GCHUNK


_K = 8
_NGRP = _NG // _K


def _sc_gather_body(idx_hbm, table_hbm, out_hbm, idx_v, rows_v, gsem, osem):
    nc = 2
    wid = lax.axis_index("s") * nc + lax.axis_index("c")
    pltpu.sync_copy(idx_hbm.at[pl.ds(wid * _NG, _NG)], idx_v)
    row0 = wid * _ROWS_PER_W

    def g_start(j, buf):
        pltpu.async_copy(table_hbm.at[idx_v.at[j]], rows_v.at[buf], gsem)

    def g_drain1():
        pltpu.make_async_copy(out_hbm.at[pl.ds(0, _GCHUNK)], rows_v.at[0],
                              gsem).wait()

    def o_start(j, buf):
        pltpu.async_copy(rows_v.at[buf],
                         out_hbm.at[pl.ds(row0 + j * _GCHUNK, _GCHUNK)], osem)

    def o_drain1():
        pltpu.make_async_copy(rows_v.at[0], out_hbm.at[pl.ds(0, _GCHUNK)],
                              osem).wait()

    for t in range(_K):
        g_start(t, t)
    for t in range(_K):
        g_drain1()
    for t in range(_K):
        g_start(_K + t, _K + t)
    for t in range(_K):
        o_start(t, t)

    def grp(g, _):
        s = (g % 2) * _K
        s2 = ((g + 1) % 2) * _K
        for t in range(_K):
            g_drain1()
        for t in range(_K):
            o_drain1()
        for t in range(_K):
            g_start((g + 1) * _K + t, s2 + t)
        for t in range(_K):
            o_start(g * _K + t, s + t)
        return _

    lax.fori_loop(1, _NGRP - 1, grp, None)

    gl = _NGRP - 1
    s = (gl % 2) * _K
    for t in range(_K):
        g_drain1()
    for t in range(_K):
        o_drain1()
    for t in range(_K):
        o_start(gl * _K + t, s + t)
    for t in range(_K):
        o_drain1()


@functools.partial(
    pl.kernel,
    mesh=plsc.VectorSubcoreMesh(core_axis_name="c", subcore_axis_name="s"),
    out_type=jax.ShapeDtypeStruct((_ROWS, D), jnp.float32),
    scratch_types=[
        pltpu.VMEM((_NG, _GCHUNK), jnp.int32),
        pltpu.VMEM((2 * _K, _GCHUNK, D), jnp.float32),
        pltpu.SemaphoreType.DMA,
        pltpu.SemaphoreType.DMA,
    ],
    compiler_params=pltpu.CompilerParams(use_tc_tiling_on_sc=False),
)
def _sc_gather(idx_hbm, table_hbm, out_hbm, idx_v, rows_v, gsem, osem):
    _sc_gather_body(idx_hbm, table_hbm, out_hbm, idx_v, rows_v, gsem, osem)


_BM = 2048


def _tc_mlp_body(x_ref, w_ref, b_ref, g_ref, be_ref, o_ref):
    x = x_ref[...]
    y = jnp.dot(x, w_ref[...], preferred_element_type=jnp.float32) + b_ref[...]
    mu = jnp.mean(y, axis=-1, keepdims=True)
    var = jnp.mean((y - mu) * (y - mu), axis=-1, keepdims=True)
    y = (y - mu) * lax.rsqrt(var + 1e-5)
    y = y * g_ref[...] + be_ref[...]
    o_ref[...] = y * 0.5 * (1.0 + lax.erf(y * (1.0 / math.sqrt(2.0))))


def _tc_mlp(x, W, b, gamma, beta):
    grid = (B // _BM,)
    return pl.pallas_call(
        _tc_mlp_body,
        grid=grid,
        in_specs=[
            pl.BlockSpec((_BM, F * D), lambda i: (i, 0)),
            pl.BlockSpec((F * D, D_OUT), lambda i: (0, 0)),
            pl.BlockSpec((1, D_OUT), lambda i: (0, 0)),
            pl.BlockSpec((1, D_OUT), lambda i: (0, 0)),
            pl.BlockSpec((1, D_OUT), lambda i: (0, 0)),
        ],
        out_specs=pl.BlockSpec((_BM, D_OUT), lambda i: (i, 0)),
        out_shape=jax.ShapeDtypeStruct((B, D_OUT), jnp.float32),
    )(x, W, b, gamma, beta)


def kernel(batch_factors, emb, W, b, gamma, beta):
    offs = (jnp.arange(F, dtype=jnp.int32) * VOCAB)[None, :]
    flat_idx = (batch_factors + offs).reshape(_ROWS // _GCHUNK, _GCHUNK)
    table = emb.reshape(F * VOCAB, D)
    gathered = _sc_gather(flat_idx, table)
    x = gathered.reshape(B, F * D)
    out = _tc_mlp(x, W, b.reshape(1, D_OUT), gamma.reshape(1, D_OUT),
                  beta.reshape(1, D_OUT))
    return (out, jnp.ones((F,), dtype=jnp.float32))


# speedup vs baseline: 12.5127x; 1.5323x over previous
import functools
import math

import jax
import jax.numpy as jnp
from jax import lax
from jax.experimental import pallas as pl
from jax.experimental.pallas import tpu as pltpu
from jax.experimental.pallas import tpu_sc as plsc

F = 26
VOCAB = 100000
D = 32
D_OUT = 128
B = 16384

_NW = 32
_BSUB = 2048
_NB = B // _BSUB


def _sc_gather_body(idxT_hbm, table_hbm, out_hbm, vec_v, idx_v, out_v,
                    isem, osem):
    nc = 2
    d = lax.axis_index("s") * nc + lax.axis_index("c")

    def idx_drain1():
        pltpu.make_async_copy(idxT_hbm.at[0, pl.ds(0, _BSUB)], idx_v.at[0],
                              isem).wait()

    def out_drain1():
        pltpu.make_async_copy(out_v.at[0], out_hbm.at[0, pl.ds(0, _BSUB)],
                              osem).wait()

    def per_field(f, carry):
        pltpu.async_copy(idxT_hbm.at[f, pl.ds(0, _BSUB)], idx_v.at[0], isem)
        pltpu.sync_copy(table_hbm.at[f, d], vec_v)
        row = f * D + d
        for c in range(_NB):
            t = c % 2
            idx_drain1()
            if c + 1 < _NB:
                pltpu.async_copy(idxT_hbm.at[f, pl.ds((c + 1) * _BSUB, _BSUB)],
                                 idx_v.at[(c + 1) % 2], isem)
            if c >= 2:
                out_drain1()

            def gidx(i, _, t=t):
                base = i * 64
                for u in range(4):
                    sl = pl.ds(base + u * 16, 16)
                    out_v[t, sl] = plsc.load_gather(vec_v, [idx_v[t, sl]])
                return _

            lax.fori_loop(0, _BSUB // 64, gidx, None)
            pltpu.async_copy(out_v.at[t],
                             out_hbm.at[row, pl.ds(c * _BSUB, _BSUB)], osem)
        out_drain1()
        out_drain1()
        return carry

    lax.fori_loop(0, F, per_field, None)


@functools.partial(
    pl.kernel,
    mesh=plsc.VectorSubcoreMesh(core_axis_name="c", subcore_axis_name="s"),
    out_type=jax.ShapeDtypeStruct((F * D, B), jnp.float32),
    scratch_types=[
        pltpu.VMEM((VOCAB,), jnp.float32),
        pltpu.VMEM((2, _BSUB), jnp.int32),
        pltpu.VMEM((2, _BSUB), jnp.float32),
        pltpu.SemaphoreType.DMA,
        pltpu.SemaphoreType.DMA,
    ],
    compiler_params=pltpu.CompilerParams(use_tc_tiling_on_sc=False,
                                         needs_layout_passes=False),
)
def _sc_gather(idxT_hbm, table_hbm, out_hbm, vec_v, idx_v, out_v, isem, osem):
    _sc_gather_body(idxT_hbm, table_hbm, out_hbm, vec_v, idx_v, out_v,
                    isem, osem)


_BM = 2048


def _tc_mlp_body(x_ref, w_ref, b_ref, g_ref, be_ref, o_ref):
    y = lax.dot_general(x_ref[...], w_ref[...], (((0,), (0,)), ((), ())),
                        preferred_element_type=jnp.float32) + b_ref[...]
    mu = jnp.mean(y, axis=-1, keepdims=True)
    var = jnp.mean((y - mu) * (y - mu), axis=-1, keepdims=True)
    y = (y - mu) * lax.rsqrt(var + 1e-5)
    y = y * g_ref[...] + be_ref[...]
    o_ref[...] = y * 0.5 * (1.0 + lax.erf(y * (1.0 / math.sqrt(2.0))))


def _tc_mlp(xT, W, b, gamma, beta):
    grid = (B // _BM,)
    return pl.pallas_call(
        _tc_mlp_body,
        grid=grid,
        in_specs=[
            pl.BlockSpec((F * D, _BM), lambda i: (0, i)),
            pl.BlockSpec((F * D, D_OUT), lambda i: (0, 0)),
            pl.BlockSpec((1, D_OUT), lambda i: (0, 0)),
            pl.BlockSpec((1, D_OUT), lambda i: (0, 0)),
            pl.BlockSpec((1, D_OUT), lambda i: (0, 0)),
        ],
        out_specs=pl.BlockSpec((_BM, D_OUT), lambda i: (i, 0)),
        out_shape=jax.ShapeDtypeStruct((B, D_OUT), jnp.float32),
    )(xT, W, b, gamma, beta)


def kernel(batch_factors, emb, W, b, gamma, beta):
    idxT = batch_factors.T
    embT = jnp.swapaxes(emb, 1, 2)
    xT = _sc_gather(idxT, embT)
    out = _tc_mlp(xT, W, b.reshape(1, D_OUT), gamma.reshape(1, D_OUT),
                  beta.reshape(1, D_OUT))
    return (out, jnp.ones((F,), dtype=jnp.float32))


# speedup vs baseline: 27.3091x; 2.1825x over previous
import functools
import math

import jax
import jax.numpy as jnp
from jax import lax
from jax.experimental import pallas as pl
from jax.experimental.pallas import tpu as pltpu
from jax.experimental.pallas import tpu_sc as plsc

F = 26
VOCAB = 100000
D = 32
D_OUT = 128
B = 16384

_NW = 32
_BSUB = 2048
_NB = B // _BSUB


def _sc_gather_body(idxT_hbm, table_hbm, out_hbm, vec_v, idx_v, out_v,
                    isem, osem):
    nc = 2
    d = lax.axis_index("s") * nc + lax.axis_index("c")

    def idx_drain1():
        pltpu.make_async_copy(idxT_hbm.at[0, pl.ds(0, _BSUB)], idx_v.at[0],
                              isem).wait()

    def out_drain1():
        pltpu.make_async_copy(out_v.at[0], out_hbm.at[0, pl.ds(0, _BSUB)],
                              osem).wait()

    def per_field(f, carry):
        pltpu.async_copy(idxT_hbm.at[f, pl.ds(0, _BSUB)], idx_v.at[0], isem)
        pltpu.sync_copy(table_hbm.at[f, d], vec_v)
        row = f * D + d
        for c in range(_NB):
            t = c % 2
            idx_drain1()
            if c + 1 < _NB:
                pltpu.async_copy(idxT_hbm.at[f, pl.ds((c + 1) * _BSUB, _BSUB)],
                                 idx_v.at[(c + 1) % 2], isem)
            if c >= 2:
                out_drain1()

            def gidx(i, _, t=t):
                base = i * 64
                for u in range(4):
                    sl = pl.ds(base + u * 16, 16)
                    out_v[t, sl] = plsc.load_gather(vec_v, [idx_v[t, sl]])
                return _

            lax.fori_loop(0, _BSUB // 64, gidx, None)
            pltpu.async_copy(out_v.at[t],
                             out_hbm.at[row, pl.ds(c * _BSUB, _BSUB)], osem)
        out_drain1()
        out_drain1()
        return carry

    lax.fori_loop(0, F, per_field, None)


@functools.partial(
    pl.kernel,
    mesh=plsc.VectorSubcoreMesh(core_axis_name="c", subcore_axis_name="s"),
    out_type=jax.ShapeDtypeStruct((F * D, B), jnp.float32),
    scratch_types=[
        pltpu.VMEM((VOCAB,), jnp.float32),
        pltpu.VMEM((2, _BSUB), jnp.int32),
        pltpu.VMEM((2, _BSUB), jnp.float32),
        pltpu.SemaphoreType.DMA,
        pltpu.SemaphoreType.DMA,
    ],
    compiler_params=pltpu.CompilerParams(use_tc_tiling_on_sc=True,
                                         needs_layout_passes=False),
)
def _sc_gather(idxT_hbm, table_hbm, out_hbm, vec_v, idx_v, out_v, isem, osem):
    _sc_gather_body(idxT_hbm, table_hbm, out_hbm, vec_v, idx_v, out_v,
                    isem, osem)


_BM = 2048


def _tc_mlp_body(x_ref, w_ref, b_ref, g_ref, be_ref, o_ref):
    y = lax.dot_general(x_ref[...], w_ref[...], (((0,), (0,)), ((), ())),
                        preferred_element_type=jnp.float32) + b_ref[...]
    mu = jnp.mean(y, axis=-1, keepdims=True)
    var = jnp.mean((y - mu) * (y - mu), axis=-1, keepdims=True)
    y = (y - mu) * lax.rsqrt(var + 1e-5)
    y = y * g_ref[...] + be_ref[...]
    o_ref[...] = y * 0.5 * (1.0 + lax.erf(y * (1.0 / math.sqrt(2.0))))


def _tc_mlp(xT, W, b, gamma, beta):
    grid = (B // _BM,)
    return pl.pallas_call(
        _tc_mlp_body,
        grid=grid,
        in_specs=[
            pl.BlockSpec((F * D, _BM), lambda i: (0, i)),
            pl.BlockSpec((F * D, D_OUT), lambda i: (0, 0)),
            pl.BlockSpec((1, D_OUT), lambda i: (0, 0)),
            pl.BlockSpec((1, D_OUT), lambda i: (0, 0)),
            pl.BlockSpec((1, D_OUT), lambda i: (0, 0)),
        ],
        out_specs=pl.BlockSpec((_BM, D_OUT), lambda i: (i, 0)),
        out_shape=jax.ShapeDtypeStruct((B, D_OUT), jnp.float32),
    )(xT, W, b, gamma, beta)


def kernel(batch_factors, emb, W, b, gamma, beta):
    idxT = batch_factors.T
    embT = jnp.swapaxes(emb, 1, 2)
    xT = _sc_gather(idxT, embT)
    out = _tc_mlp(xT, W, b.reshape(1, D_OUT), gamma.reshape(1, D_OUT),
                  beta.reshape(1, D_OUT))
    return (out, jnp.ones((F,), dtype=jnp.float32))
